# baseline (device time: 7237 ns/iter reference)
import jax
import jax.numpy as jnp
from jax import lax
from jax.experimental import pallas as pl
from jax.experimental.pallas import tpu as pltpu


def kernel(x):
    m, n = x.shape

    def body(x_ref, out_ref, send_row, send_col, halo_row, halo_col,
             send_sems, recv_sems):
        my_x = lax.axis_index("x")
        my_y = lax.axis_index("y")

        barrier_sem = pltpu.get_barrier_semaphore()
        pl.semaphore_signal(barrier_sem, inc=1, device_id=(1 - my_x, my_y),
                            device_id_type=pl.DeviceIdType.MESH)
        pl.semaphore_signal(barrier_sem, inc=1, device_id=(my_x, 1 - my_y),
                            device_id_type=pl.DeviceIdType.MESH)
        pl.semaphore_wait(barrier_sem, 2)

        xv = x_ref[:, :]

        send_row[:, :] = jnp.where(my_x == 0, xv[m - 1:m, :], xv[0:1, :])
        send_col[:, :] = jnp.where(my_y == 0, xv[:, n - 1:n], xv[:, 0:1])

        row_rdma = pltpu.make_async_remote_copy(
            src_ref=send_row, dst_ref=halo_row,
            send_sem=send_sems.at[0], recv_sem=recv_sems.at[0],
            device_id=(1 - my_x, my_y), device_id_type=pl.DeviceIdType.MESH,
        )
        col_rdma = pltpu.make_async_remote_copy(
            src_ref=send_col, dst_ref=halo_col,
            send_sem=send_sems.at[1], recv_sem=recv_sems.at[1],
            device_id=(my_x, 1 - my_y), device_id_type=pl.DeviceIdType.MESH,
        )
        row_rdma.start()
        col_rdma.start()

        stencil = xv

        ri = lax.broadcasted_iota(jnp.int32, (m, n), 0)
        ci = lax.broadcasted_iota(jnp.int32, (m, n), 1)
        boundary = (
            ((my_x == 0) & (ri == 0)) | ((my_x == 1) & (ri == m - 1))
            | ((my_y == 0) & (ci == 0)) | ((my_y == 1) & (ci == n - 1))
        )

        r_idx = jnp.where(my_x == 1, 0, m - 1)
        c_idx = jnp.where(my_y == 1, 0, n - 1)
        row_rdma.wait_recv()
        col_rdma.wait_recv()
        patch = (
            jnp.where(ri == r_idx, 0.125 * halo_row[:, :], 0.0)
            + jnp.where(ci == c_idx, 0.125 * halo_col[:, :], 0.0)
        )
        out_ref[:, :] = jnp.where(boundary, xv, stencil + patch)

        row_rdma.wait_send()
        col_rdma.wait_send()

    return pl.pallas_call(
        body,
        out_shape=jax.ShapeDtypeStruct((m, n), jnp.float32),
        in_specs=[pl.BlockSpec(memory_space=pltpu.VMEM)],
        out_specs=pl.BlockSpec(memory_space=pltpu.VMEM),
        scratch_shapes=[
            pltpu.VMEM((1, n), jnp.float32),
            pltpu.VMEM((m, 1), jnp.float32),
            pltpu.VMEM((1, n), jnp.float32),
            pltpu.VMEM((m, 1), jnp.float32),
            pltpu.SemaphoreType.DMA((2,)),
            pltpu.SemaphoreType.DMA((2,)),
        ],
        compiler_params=pltpu.CompilerParams(collective_id=0),
    )(x)


# device time: 7211 ns/iter; 1.0036x vs baseline; 1.0036x over previous
import jax
import jax.numpy as jnp
from jax import lax
from jax.experimental import pallas as pl
from jax.experimental.pallas import tpu as pltpu


def kernel(x):
    m, n = x.shape

    def body(x_ref, out_ref, send_row, send_col, halo_row, halo_col,
             send_sems, recv_sems):
        my_x = lax.axis_index("x")
        my_y = lax.axis_index("y")

        barrier_sem = pltpu.get_barrier_semaphore()
        pl.semaphore_signal(barrier_sem, inc=1, device_id=(1 - my_x, my_y),
                            device_id_type=pl.DeviceIdType.MESH)
        pl.semaphore_signal(barrier_sem, inc=1, device_id=(my_x, 1 - my_y),
                            device_id_type=pl.DeviceIdType.MESH)
        pl.semaphore_wait(barrier_sem, 2)

        xv = x_ref[:, :]

        send_row[:, :] = jnp.where(my_x == 0, xv[m - 1:m, :], xv[0:1, :])
        send_col[:, :] = jnp.where(my_y == 0, xv[:, n - 1:n], xv[:, 0:1])

        col_rdma = pltpu.make_async_remote_copy(
            src_ref=send_col, dst_ref=halo_col,
            send_sem=send_sems.at[1], recv_sem=recv_sems.at[1],
            device_id=(my_x, 1 - my_y), device_id_type=pl.DeviceIdType.MESH,
        )
        col_rdma.start()

        zrow = jnp.zeros((1, n), jnp.float32)
        zcol = jnp.zeros((m, 1), jnp.float32)
        up = jnp.concatenate([zrow, xv[:-1, :]], axis=0)
        down = jnp.concatenate([xv[1:, :], zrow], axis=0)
        left = jnp.concatenate([zcol, xv[:, :-1]], axis=1)
        right = jnp.concatenate([xv[:, 1:], zcol], axis=1)
        stencil = 0.5 * xv + 0.125 * (up + down + left + right)

        ri = lax.broadcasted_iota(jnp.int32, (m, n), 0)
        ci = lax.broadcasted_iota(jnp.int32, (m, n), 1)
        boundary = (
            ((my_x == 0) & (ri == 0)) | ((my_x == 1) & (ri == m - 1))
            | ((my_y == 0) & (ci == 0)) | ((my_y == 1) & (ci == n - 1))
        )

        r_idx = jnp.where(my_x == 1, 0, m - 1)
        c_idx = jnp.where(my_y == 1, 0, n - 1)
        col_rdma.wait_recv()
        patch = jnp.where(ci == c_idx, 0.125 * halo_col[:, :], 0.0)
        out_ref[:, :] = jnp.where(boundary, xv, stencil + patch)

        col_rdma.wait_send()

    return pl.pallas_call(
        body,
        out_shape=jax.ShapeDtypeStruct((m, n), jnp.float32),
        in_specs=[pl.BlockSpec(memory_space=pltpu.VMEM)],
        out_specs=pl.BlockSpec(memory_space=pltpu.VMEM),
        scratch_shapes=[
            pltpu.VMEM((1, n), jnp.float32),
            pltpu.VMEM((m, 1), jnp.float32),
            pltpu.VMEM((1, n), jnp.float32),
            pltpu.VMEM((m, 1), jnp.float32),
            pltpu.SemaphoreType.DMA((2,)),
            pltpu.SemaphoreType.DMA((2,)),
        ],
        compiler_params=pltpu.CompilerParams(collective_id=0),
    )(x)


# device time: 6612 ns/iter; 1.0945x vs baseline; 1.0906x over previous
import jax
import jax.numpy as jnp
from jax import lax
from jax.experimental import pallas as pl
from jax.experimental.pallas import tpu as pltpu


def kernel(x):
    m, n = x.shape

    def body(x_ref, out_ref, send_row, send_colt, halo_row, halo_colt,
             send_sems, recv_sems):
        my_x = lax.axis_index("x")
        my_y = lax.axis_index("y")

        barrier_sem = pltpu.get_barrier_semaphore()
        pl.semaphore_signal(barrier_sem, inc=1, device_id=(1 - my_x, my_y),
                            device_id_type=pl.DeviceIdType.MESH)
        pl.semaphore_signal(barrier_sem, inc=1, device_id=(my_x, 1 - my_y),
                            device_id_type=pl.DeviceIdType.MESH)

        xv = x_ref[:, :]

        send_row[:, :] = jnp.where(my_x == 0, xv[m - 1:m, :], xv[0:1, :])
        t_lo = jnp.transpose(xv[:, 0:8])
        t_hi = jnp.transpose(xv[:, n - 8:n])
        send_colt[:, :] = jnp.where(my_y == 0, t_hi, t_lo)

        pl.semaphore_wait(barrier_sem, 2)

        row_rdma = pltpu.make_async_remote_copy(
            src_ref=send_row, dst_ref=halo_row,
            send_sem=send_sems.at[0], recv_sem=recv_sems.at[0],
            device_id=(1 - my_x, my_y), device_id_type=pl.DeviceIdType.MESH,
        )
        col_rdma = pltpu.make_async_remote_copy(
            src_ref=send_colt, dst_ref=halo_colt,
            send_sem=send_sems.at[1], recv_sem=recv_sems.at[1],
            device_id=(my_x, 1 - my_y), device_id_type=pl.DeviceIdType.MESH,
        )
        row_rdma.start()
        col_rdma.start()

        zrow = jnp.zeros((1, n), jnp.float32)
        zcol = jnp.zeros((m, 1), jnp.float32)
        up = jnp.concatenate([zrow, xv[:-1, :]], axis=0)
        down = jnp.concatenate([xv[1:, :], zrow], axis=0)
        left = jnp.concatenate([zcol, xv[:, :-1]], axis=1)
        right = jnp.concatenate([xv[:, 1:], zcol], axis=1)
        stencil = 0.5 * xv + 0.125 * (up + down + left + right)

        ri = lax.broadcasted_iota(jnp.int32, (m, n), 0)
        ci = lax.broadcasted_iota(jnp.int32, (m, n), 1)
        boundary = (
            ((my_x == 0) & (ri == 0)) | ((my_x == 1) & (ri == m - 1))
            | ((my_y == 0) & (ci == 0)) | ((my_y == 1) & (ci == n - 1))
        )

        r_idx = jnp.where(my_x == 1, 0, m - 1)
        c_idx = jnp.where(my_y == 1, 0, n - 1)
        row_rdma.wait_recv()
        col_rdma.wait_recv()
        colv = jnp.transpose(halo_colt[:, :])
        halo_col = jnp.where(my_y == 0, colv[:, 0:1], colv[:, 7:8])
        patch = (
            jnp.where(ri == r_idx, 0.125 * halo_row[:, :], 0.0)
            + jnp.where(ci == c_idx, 0.125 * halo_col, 0.0)
        )
        out_ref[:, :] = jnp.where(boundary, xv, stencil + patch)

        row_rdma.wait_send()
        col_rdma.wait_send()

    return pl.pallas_call(
        body,
        out_shape=jax.ShapeDtypeStruct((m, n), jnp.float32),
        in_specs=[pl.BlockSpec(memory_space=pltpu.VMEM)],
        out_specs=pl.BlockSpec(memory_space=pltpu.VMEM),
        scratch_shapes=[
            pltpu.VMEM((1, n), jnp.float32),
            pltpu.VMEM((8, n), jnp.float32),
            pltpu.VMEM((1, n), jnp.float32),
            pltpu.VMEM((8, n), jnp.float32),
            pltpu.SemaphoreType.DMA((2,)),
            pltpu.SemaphoreType.DMA((2,)),
        ],
        compiler_params=pltpu.CompilerParams(collective_id=0),
    )(x)


# device time: 6556 ns/iter; 1.1039x vs baseline; 1.0085x over previous
import jax
import jax.numpy as jnp
from jax import lax
from jax.experimental import pallas as pl
from jax.experimental.pallas import tpu as pltpu


def kernel(x):
    m, n = x.shape

    def body(x_ref, out_ref, send_row, send_colt, halo_row, halo_colt,
             send_sems, recv_sems):
        my_x = lax.axis_index("x")
        my_y = lax.axis_index("y")

        barrier_sem = pltpu.get_barrier_semaphore()
        pl.semaphore_signal(barrier_sem, inc=1, device_id=(1 - my_x, my_y),
                            device_id_type=pl.DeviceIdType.MESH)
        pl.semaphore_signal(barrier_sem, inc=1, device_id=(my_x, 1 - my_y),
                            device_id_type=pl.DeviceIdType.MESH)

        xv = x_ref[:, :]

        send_row[:, :] = jnp.where(my_x == 0, xv[m - 1:m, :], xv[0:1, :])
        t_lo = jnp.transpose(xv[:, 0:8])
        t_hi = jnp.transpose(xv[:, n - 8:n])
        send_colt[:, :] = jnp.where(my_y == 0, t_hi, t_lo)

        zrow = jnp.zeros((1, n), jnp.float32)
        zcol = jnp.zeros((m, 1), jnp.float32)
        up = jnp.concatenate([zrow, xv[:-1, :]], axis=0)
        down = jnp.concatenate([xv[1:, :], zrow], axis=0)
        left = jnp.concatenate([zcol, xv[:, :-1]], axis=1)
        right = jnp.concatenate([xv[:, 1:], zcol], axis=1)
        stencil = 0.5 * xv + 0.125 * (up + down + left + right)

        ri = lax.broadcasted_iota(jnp.int32, (m, n), 0)
        ci = lax.broadcasted_iota(jnp.int32, (m, n), 1)
        boundary = (
            ((my_x == 0) & (ri == 0)) | ((my_x == 1) & (ri == m - 1))
            | ((my_y == 0) & (ci == 0)) | ((my_y == 1) & (ci == n - 1))
        )

        pl.semaphore_wait(barrier_sem, 2)

        row_rdma = pltpu.make_async_remote_copy(
            src_ref=send_row, dst_ref=halo_row,
            send_sem=send_sems.at[0], recv_sem=recv_sems.at[0],
            device_id=(1 - my_x, my_y), device_id_type=pl.DeviceIdType.MESH,
        )
        col_rdma = pltpu.make_async_remote_copy(
            src_ref=send_colt, dst_ref=halo_colt,
            send_sem=send_sems.at[1], recv_sem=recv_sems.at[1],
            device_id=(my_x, 1 - my_y), device_id_type=pl.DeviceIdType.MESH,
        )
        row_rdma.start()
        col_rdma.start()

        r_idx = jnp.where(my_x == 1, 0, m - 1)
        c_idx = jnp.where(my_y == 1, 0, n - 1)
        row_rdma.wait_recv()
        col_rdma.wait_recv()
        colv = jnp.transpose(halo_colt[:, :])
        halo_col = jnp.where(my_y == 0, colv[:, 0:1], colv[:, 7:8])
        patch = (
            jnp.where(ri == r_idx, 0.125 * halo_row[:, :], 0.0)
            + jnp.where(ci == c_idx, 0.125 * halo_col, 0.0)
        )
        out_ref[:, :] = jnp.where(boundary, xv, stencil + patch)

        row_rdma.wait_send()
        col_rdma.wait_send()

    return pl.pallas_call(
        body,
        out_shape=jax.ShapeDtypeStruct((m, n), jnp.float32),
        in_specs=[pl.BlockSpec(memory_space=pltpu.VMEM)],
        out_specs=pl.BlockSpec(memory_space=pltpu.VMEM),
        scratch_shapes=[
            pltpu.VMEM((1, n), jnp.float32),
            pltpu.VMEM((8, n), jnp.float32),
            pltpu.VMEM((1, n), jnp.float32),
            pltpu.VMEM((8, n), jnp.float32),
            pltpu.SemaphoreType.DMA((2,)),
            pltpu.SemaphoreType.DMA((2,)),
        ],
        compiler_params=pltpu.CompilerParams(collective_id=0),
    )(x)


# device time: 5464 ns/iter; 1.3245x vs baseline; 1.1999x over previous
import jax
import jax.numpy as jnp
from jax import lax
from jax.experimental import pallas as pl
from jax.experimental.pallas import tpu as pltpu


def kernel(x):
    m, n = x.shape

    def body(x_ref, out_ref, send_row, send_colt, halo_row, halo_colt,
             send_sems, recv_sems):
        my_x = lax.axis_index("x")
        my_y = lax.axis_index("y")

        barrier_sem = pltpu.get_barrier_semaphore()
        pl.semaphore_signal(barrier_sem, inc=1)
        pl.semaphore_wait(barrier_sem, 1)

        xv = x_ref[:, :]

        send_row[:, :] = jnp.where(my_x == 0, xv[m - 1:m, :], xv[0:1, :])
        row_rdma = pltpu.make_async_remote_copy(
            src_ref=send_row, dst_ref=halo_row,
            send_sem=send_sems.at[0], recv_sem=recv_sems.at[0],
            device_id=(1 - my_x, my_y), device_id_type=pl.DeviceIdType.MESH,
        )
        row_rdma.start()
        t_lo = jnp.transpose(xv[:, 0:8])
        t_hi = jnp.transpose(xv[:, n - 8:n])
        send_colt[:, :] = jnp.where(my_y == 0, t_hi, t_lo)
        col_rdma = pltpu.make_async_remote_copy(
            src_ref=send_colt, dst_ref=halo_colt,
            send_sem=send_sems.at[1], recv_sem=recv_sems.at[1],
            device_id=(my_x, 1 - my_y), device_id_type=pl.DeviceIdType.MESH,
        )
        col_rdma.start()

        zrow = jnp.zeros((1, n), jnp.float32)
        zcol = jnp.zeros((m, 1), jnp.float32)
        up = jnp.concatenate([zrow, xv[:-1, :]], axis=0)
        down = jnp.concatenate([xv[1:, :], zrow], axis=0)
        left = jnp.concatenate([zcol, xv[:, :-1]], axis=1)
        right = jnp.concatenate([xv[:, 1:], zcol], axis=1)
        stencil = 0.5 * xv + 0.125 * (up + down + left + right)

        ri = lax.broadcasted_iota(jnp.int32, (m, n), 0)
        ci = lax.broadcasted_iota(jnp.int32, (m, n), 1)
        boundary = (
            ((my_x == 0) & (ri == 0)) | ((my_x == 1) & (ri == m - 1))
            | ((my_y == 0) & (ci == 0)) | ((my_y == 1) & (ci == n - 1))
        )


        r_idx = jnp.where(my_x == 1, 0, m - 1)
        c_idx = jnp.where(my_y == 1, 0, n - 1)
        row_rdma.wait_recv()
        col_rdma.wait_recv()
        colv = jnp.transpose(halo_colt[:, :])
        halo_col = jnp.where(my_y == 0, colv[:, 0:1], colv[:, 7:8])
        patch = (
            jnp.where(ri == r_idx, 0.125 * halo_row[:, :], 0.0)
            + jnp.where(ci == c_idx, 0.125 * halo_col, 0.0)
        )
        out_ref[:, :] = jnp.where(boundary, xv, stencil + patch)

        row_rdma.wait_send()
        col_rdma.wait_send()

    return pl.pallas_call(
        body,
        out_shape=jax.ShapeDtypeStruct((m, n), jnp.float32),
        in_specs=[pl.BlockSpec(memory_space=pltpu.VMEM)],
        out_specs=pl.BlockSpec(memory_space=pltpu.VMEM),
        scratch_shapes=[
            pltpu.VMEM((1, n), jnp.float32),
            pltpu.VMEM((8, n), jnp.float32),
            pltpu.VMEM((1, n), jnp.float32),
            pltpu.VMEM((8, n), jnp.float32),
            pltpu.SemaphoreType.DMA((2,)),
            pltpu.SemaphoreType.DMA((2,)),
        ],
        compiler_params=pltpu.CompilerParams(collective_id=0),
    )(x)
